# Initial kernel scaffold; baseline (speedup 1.0000x reference)
import jax
import jax.numpy as jnp
from jax.experimental import pallas as pl


def kernel(nodes, features, adj, W1, W2, Wc):
    raise NotImplementedError("write your pallas kernel here")


# speedup vs baseline: 8.0964x; 8.0964x over previous
import functools

import jax
import jax.numpy as jnp
from jax import lax
from jax.experimental import pallas as pl
from jax.experimental.pallas import tpu as pltpu
from jax.experimental.pallas import tpu_sc as plsc

N = 100000
D = 128
B = 16384
H1 = 128
H2 = 128
C = 40

NC, NS = 2, 16
NW = NC * NS
BPW = B // NW
CH = 128
NCHUNK = BPW // CH
LANES = 16


def _sc_aggregate(nodes, adj0, adj1, adj2, features):
    mesh = plsc.VectorSubcoreMesh(
        core_axis_name="c", subcore_axis_name="s",
        num_cores=NC, num_subcores=NS)

    @functools.partial(
        pl.kernel,
        out_type=jax.ShapeDtypeStruct((NW * 4 * BPW, D), jnp.float32),
        mesh=mesh,
        scratch_types=[
            pltpu.VMEM((BPW,), jnp.int32),
            pltpu.VMEM((BPW,), jnp.int32),
            pltpu.VMEM((BPW,), jnp.int32),
            pltpu.VMEM((BPW,), jnp.int32),
            pltpu.VMEM((BPW,), jnp.int32),
            pltpu.VMEM((BPW,), jnp.int32),
            pltpu.VMEM((BPW,), jnp.int32),
            pltpu.VMEM((CH, D), jnp.float32),
            pltpu.VMEM((CH, D), jnp.float32),
            pltpu.VMEM((CH, D), jnp.float32),
            pltpu.VMEM((CH, D), jnp.float32),
            pltpu.SemaphoreType.DMA,
        ],
    )
    def k(nodes_hbm, adj0_hbm, adj1_hbm, adj2_hbm, feat_hbm, out_hbm,
          nodes_v, s0, s1, s2, g0, g1, g2, b0, b1, b2, b3, sem):
        wid = lax.axis_index("s") * NC + lax.axis_index("c")
        base = wid * BPW
        pltpu.sync_copy(nodes_hbm.at[pl.ds(base, BPW)], nodes_v)
        c0 = pltpu.async_copy(adj0_hbm.at[nodes_v], s0, sem)
        c1 = pltpu.async_copy(adj1_hbm.at[nodes_v], s1, sem)
        c2 = pltpu.async_copy(adj2_hbm.at[nodes_v], s2, sem)
        c0.wait(); c1.wait(); c2.wait()
        for j, S in enumerate((s0, s1, s2, nodes_v)):
            d0 = pltpu.async_copy(adj0_hbm.at[S], g0, sem)
            d1 = pltpu.async_copy(adj1_hbm.at[S], g1, sem)
            d2 = pltpu.async_copy(adj2_hbm.at[S], g2, sem)
            d0.wait(); d1.wait(); d2.wait()
            for t in range(NCHUNK):
                sl = pl.ds(t * CH, CH)
                e0 = pltpu.async_copy(feat_hbm.at[g0.at[sl]], b0, sem)
                e1 = pltpu.async_copy(feat_hbm.at[g1.at[sl]], b1, sem)
                e2 = pltpu.async_copy(feat_hbm.at[g2.at[sl]], b2, sem)
                e3 = pltpu.async_copy(feat_hbm.at[S.at[sl]], b3, sem)
                e0.wait(); e1.wait(); e2.wait(); e3.wait()

                def body(i, carry):
                    for kk in range(D // LANES):
                        cs = pl.ds(kk * LANES, LANES)
                        b0[i, cs] = b0[i, cs] + b1[i, cs] + b2[i, cs] + b3[i, cs]
                    return carry
                lax.fori_loop(0, CH, body, 0)
                row0 = (wid * 4 + j) * BPW + t * CH
                pltpu.sync_copy(b0, out_hbm.at[pl.ds(row0, CH)])

    return k(nodes, adj0, adj1, adj2, features)


def _tc_dense(A, W1s, W2s, WcT):
    def body(a_ref, w1_ref, w2_ref, wc_ref, o_ref):
        a = a_ref[...]
        h = jnp.maximum(
            jnp.dot(a, w1_ref[...], preferred_element_type=jnp.float32), 0.0)
        m = (h[0 * BPW:1 * BPW] + h[1 * BPW:2 * BPW]
             + h[2 * BPW:3 * BPW] + h[3 * BPW:4 * BPW])
        h2 = jnp.maximum(
            jnp.dot(m, w2_ref[...], preferred_element_type=jnp.float32), 0.0)
        logits = jnp.dot(h2, wc_ref[...], preferred_element_type=jnp.float32)
        mx = jnp.max(logits, axis=1, keepdims=True)
        lse = jnp.log(jnp.sum(jnp.exp(logits - mx), axis=1, keepdims=True)) + mx
        o_ref[...] = logits - lse

    return pl.pallas_call(
        body,
        grid=(NW,),
        in_specs=[
            pl.BlockSpec((4 * BPW, D), lambda w: (w, 0)),
            pl.BlockSpec((D, H1), lambda w: (0, 0)),
            pl.BlockSpec((H1, H2), lambda w: (0, 0)),
            pl.BlockSpec((H2, C), lambda w: (0, 0)),
        ],
        out_specs=pl.BlockSpec((BPW, C), lambda w: (w, 0)),
        out_shape=jax.ShapeDtypeStruct((B, C), jnp.float32),
    )(A, W1s, W2s, WcT)


def kernel(nodes, features, adj, W1, W2, Wc):
    adj0, adj1, adj2 = adj[:, 0], adj[:, 1], adj[:, 2]
    A = _sc_aggregate(nodes, adj0, adj1, adj2, features)
    W1s = W1.T * 0.25
    W2s = W2.T * 0.25
    return _tc_dense(A, W1s, W2s, Wc.T)


# speedup vs baseline: 10.0167x; 1.2372x over previous
import functools

import jax
import jax.numpy as jnp
from jax import lax
from jax.experimental import pallas as pl
from jax.experimental.pallas import tpu as pltpu
from jax.experimental.pallas import tpu_sc as plsc

N = 100000
D = 128
B = 16384
H1 = 128
H2 = 128
C = 40

NC, NS = 2, 16
NW = NC * NS
BPW = B // NW
CH = 128
NCHUNK = BPW // CH
NLIST = 4
NT = NLIST * NCHUNK
LANES = 16


def _sc_aggregate(nodes, adj0, adj1, adj2, features):
    mesh = plsc.VectorSubcoreMesh(
        core_axis_name="c", subcore_axis_name="s",
        num_cores=NC, num_subcores=NS)

    id_scratch = [pltpu.VMEM((BPW,), jnp.int32) for _ in range(16)]

    @functools.partial(
        pl.kernel,
        out_type=jax.ShapeDtypeStruct((NW * NLIST * BPW, D), jnp.float32),
        mesh=mesh,
        scratch_types=id_scratch + [
            pltpu.VMEM((CH, D), jnp.float32),
            pltpu.VMEM((CH, D), jnp.float32),
            pltpu.VMEM((CH, D), jnp.float32),
            pltpu.VMEM((CH, D), jnp.float32),
            pltpu.VMEM((CH, D), jnp.float32),
            pltpu.VMEM((CH, D), jnp.float32),
            pltpu.SemaphoreType.DMA,
            pltpu.SemaphoreType.DMA,
            pltpu.SemaphoreType.DMA,
            pltpu.SemaphoreType.DMA,
        ],
    )
    def k(nodes_hbm, adj0_hbm, adj1_hbm, adj2_hbm, feat_hbm, out_hbm,
          s0, s1, s2, nv,
          g00, g01, g02, g10, g11, g12, g20, g21, g22, g30, g31, g32,
          b0, b1, b2, b3, acc0, acc1, semA, semB, semw0, semw1):
        G = [[g00, g01, g02], [g10, g11, g12], [g20, g21, g22], [g30, g31, g32]]
        acc = (acc0, acc1)
        semw = (semw0, semw1)
        wid = lax.axis_index("s") * NC + lax.axis_index("c")
        base = wid * BPW
        pltpu.sync_copy(nodes_hbm.at[pl.ds(base, BPW)], nv)
        c0 = pltpu.async_copy(adj0_hbm.at[nv], s0, semA)
        c1 = pltpu.async_copy(adj1_hbm.at[nv], s1, semB)
        c2 = pltpu.async_copy(adj2_hbm.at[nv], s2, semw0)
        c0.wait(); c1.wait(); c2.wait()
        S = (s0, s1, s2, nv)
        gd = []
        for j in range(NLIST):
            gd.append(pltpu.async_copy(adj0_hbm.at[S[j]], G[j][0], semA))
            gd.append(pltpu.async_copy(adj1_hbm.at[S[j]], G[j][1], semB))
            gd.append(pltpu.async_copy(adj2_hbm.at[S[j]], G[j][2], semw0))
        for dsc in gd:
            dsc.wait()

        def idx(task, m):
            j, t = divmod(task, NCHUNK)
            sl = pl.ds(t * CH, CH)
            src = G[j][m] if m < 3 else S[j]
            return src.at[sl]

        def issue01(task):
            return [pltpu.async_copy(feat_hbm.at[idx(task, 0)], b0, semA),
                    pltpu.async_copy(feat_hbm.at[idx(task, 1)], b1, semA)]

        def issue23(task):
            return [pltpu.async_copy(feat_hbm.at[idx(task, 2)], b2, semB),
                    pltpu.async_copy(feat_hbm.at[idx(task, 3)], b3, semB)]

        d01 = issue01(0)
        d23 = issue23(0)
        wdescs = [None, None]
        for task in range(NT):
            ac = task % 2
            if wdescs[ac] is not None:
                wdescs[ac].wait()
            for dsc in d01:
                dsc.wait()

            def pass1(i, carry, ac=ac):
                for kk in range(D // LANES):
                    cs = pl.ds(kk * LANES, LANES)
                    acc[ac][i, cs] = b0[i, cs] + b1[i, cs]
                return carry
            lax.fori_loop(0, CH, pass1, 0)
            if task + 1 < NT:
                d01 = issue01(task + 1)
            for dsc in d23:
                dsc.wait()

            def pass2(i, carry, ac=ac):
                for kk in range(D // LANES):
                    cs = pl.ds(kk * LANES, LANES)
                    acc[ac][i, cs] = acc[ac][i, cs] + b2[i, cs] + b3[i, cs]
                return carry
            lax.fori_loop(0, CH, pass2, 0)
            if task + 1 < NT:
                d23 = issue23(task + 1)
            j, t = divmod(task, NCHUNK)
            row0 = (wid * NLIST + j) * BPW + t * CH
            wdescs[ac] = pltpu.async_copy(
                acc[ac], out_hbm.at[pl.ds(row0, CH)], semw[ac])
        for wd in wdescs:
            if wd is not None:
                wd.wait()

    return k(nodes, adj0, adj1, adj2, features)


def _tc_dense(A, W1s, W2s, WcT):
    def body(a_ref, w1_ref, w2_ref, wc_ref, o_ref):
        a = a_ref[...]
        h = jnp.maximum(
            jnp.dot(a, w1_ref[...], preferred_element_type=jnp.float32), 0.0)
        m = (h[0 * BPW:1 * BPW] + h[1 * BPW:2 * BPW]
             + h[2 * BPW:3 * BPW] + h[3 * BPW:4 * BPW])
        h2 = jnp.maximum(
            jnp.dot(m, w2_ref[...], preferred_element_type=jnp.float32), 0.0)
        logits = jnp.dot(h2, wc_ref[...], preferred_element_type=jnp.float32)
        mx = jnp.max(logits, axis=1, keepdims=True)
        lse = jnp.log(jnp.sum(jnp.exp(logits - mx), axis=1, keepdims=True)) + mx
        o_ref[...] = logits - lse

    return pl.pallas_call(
        body,
        grid=(NW,),
        in_specs=[
            pl.BlockSpec((NLIST * BPW, D), lambda w: (w, 0)),
            pl.BlockSpec((D, H1), lambda w: (0, 0)),
            pl.BlockSpec((H1, H2), lambda w: (0, 0)),
            pl.BlockSpec((H2, C), lambda w: (0, 0)),
        ],
        out_specs=pl.BlockSpec((BPW, C), lambda w: (w, 0)),
        out_shape=jax.ShapeDtypeStruct((B, C), jnp.float32),
    )(A, W1s, W2s, WcT)


def kernel(nodes, features, adj, W1, W2, Wc):
    adj0, adj1, adj2 = adj[:, 0], adj[:, 1], adj[:, 2]
    A = _sc_aggregate(nodes, adj0, adj1, adj2, features)
    W1s = W1.T * 0.25
    W2s = W2.T * 0.25
    return _tc_dense(A, W1s, W2s, Wc.T)
